# baseline (speedup 1.0000x reference)
import functools

import jax
import jax.numpy as jnp
import numpy as np
from jax import lax
from jax.experimental import pallas as pl
from jax.experimental.pallas import tpu as pltpu
from jax.experimental.pallas import tpu_sc as plsc

_GRID = (480, 360, 32)
_PCR = np.array([0.0, -np.pi, -4.0, 50.0, np.pi, 2.0], dtype=np.float32)
_VOX = np.array([(_PCR[3] - _PCR[0]) / _GRID[0], (_PCR[4] - _PCR[1]) / _GRID[1],
                 (_PCR[5] - _PCR[2]) / _GRID[2]], dtype=np.float32)
_NV = 2 * _GRID[0] * _GRID[1]
_N = 200000
_NPAD = 204800
_BN = 1024
_NBLK = _NPAD // _BN
_SENT = 1 << 29
_EPS = 1e-5
_RB = 3456
_QH = 86400


def _dotT(a, b):
    return lax.dot_general(a, b, (((0,), (0,)), ((), ())),
                           preferred_element_type=jnp.float32, precision=lax.Precision.HIGHEST)


def _dot(a, b, prec=lax.Precision.HIGHEST):
    return lax.dot_general(a, b, (((1,), (0,)), ((), ())),
                           preferred_element_type=jnp.float32, precision=prec)


def _p1_body(pts_ref, vals_ref):
    i = pl.program_id(0)
    p = pts_ref[...]
    x = p[1:2, :]
    y = p[2:3, :]
    z = p[3:4, :]
    rho = jnp.sqrt(x * x + y * y)
    phi = jnp.arctan2(y, x)
    col = jax.lax.broadcasted_iota(jnp.int32, (1, _BN), 1) + i * _BN
    valid = col < _N
    vmask = valid.astype(jnp.float32)
    ones = jnp.ones((1, _BN), jnp.float32)
    zeros = jnp.zeros((1, _BN), jnp.float32)
    vch = jnp.concatenate([rho, phi, z, x, y, ones, zeros, zeros], axis=0) * vmask
    vals_ref[...] = _dotT(vch, jnp.eye(8, dtype=jnp.float32))


def _p1(pts_t):
    return pl.pallas_call(
        _p1_body,
        grid=(_NBLK,),
        in_specs=[pl.BlockSpec((8, _BN), lambda i: (0, i))],
        out_specs=pl.BlockSpec((_BN, 8), lambda i: (i, 0)),
        out_shape=jax.ShapeDtypeStruct((_NPAD, 8), jnp.float32),
    )(pts_t)


def _p2_body(pts_ref, g_ref, k_ref, fch_ref, gram_ref):
    i = pl.program_id(0)
    p = pts_ref[...]
    x = p[1:2, :]
    y = p[2:3, :]
    z = p[3:4, :]
    inten = p[4:5, :]
    rho = jnp.sqrt(x * x + y * y)
    phi = jnp.arctan2(y, x)
    k = k_ref[0]
    vp = k % _GRID[1]
    vr = (k // _GRID[1]) % _GRID[0]
    vz = jnp.clip(jnp.floor((z - _PCR[2]) / _VOX[2]).astype(jnp.int32), 0, _GRID[2] - 1)
    cr = (vr.astype(jnp.float32) + 0.5) * _VOX[0] + _PCR[0]
    cp = (vp.astype(jnp.float32) + 0.5) * _VOX[1] + _PCR[1]
    cz = (vz.astype(jnp.float32) + 0.5) * _VOX[2] + _PCR[2]
    ey8 = jnp.eye(8, dtype=jnp.float32)
    gT0 = lax.dot_general(ey8, g_ref[0], (((1,), (1,)), ((), ())),
                          preferred_element_type=jnp.float32, precision=lax.Precision.HIGHEST)
    gT1 = lax.dot_general(ey8, g_ref[1], (((1,), (1,)), ((), ())),
                          preferred_element_type=jnp.float32, precision=lax.Precision.HIGHEST)
    gT = jnp.where(k < _HALF, gT0, gT1)
    cnt = jnp.maximum(gT[5:6, :], 1.0)
    mean5 = gT[0:5, :] / cnt
    pc5 = jnp.concatenate([rho, phi, z, x, y], axis=0)
    nor = pc5 - mean5
    c2p = jnp.concatenate([rho - cr, phi - cp, z - cz], axis=0)
    col = jax.lax.broadcasted_iota(jnp.int32, (1, _BN), 1) + i * _BN
    vmask = (col < _N).astype(jnp.float32)
    ones = jnp.ones((1, _BN), jnp.float32)
    zeros = jnp.zeros((1, _BN), jnp.float32)
    fch = jnp.concatenate([pc5, inten, nor, c2p, ones, zeros], axis=0) * vmask
    fch_ref[...] = fch

    @pl.when(i == 0)
    def _():
        gram_ref[...] = jnp.zeros_like(gram_ref)

    gram_ref[...] += lax.dot_general(fch, fch, (((1,), (1,)), ((), ())),
                                     preferred_element_type=jnp.float32, precision=lax.Precision.HIGHEST)


def _p2(pts_t, gathered, keys3):
    return pl.pallas_call(
        _p2_body,
        grid=(_NBLK,),
        in_specs=[pl.BlockSpec((8, _BN), lambda i: (0, i)),
                  pl.BlockSpec((2, _BN, 8), lambda i: (0, i, 0)),
                  pl.BlockSpec((1, 1, _BN), lambda i: (i, 0, 0))],
        out_specs=[pl.BlockSpec((16, _BN), lambda i: (0, i)),
                   pl.BlockSpec((16, 16), lambda i: (0, 0))],
        out_shape=[jax.ShapeDtypeStruct((16, _NPAD), jnp.float32),
                   jax.ShapeDtypeStruct((16, 16), jnp.float32)],
    )(pts_t, gathered, keys3)


def _p3_body(fch_ref, w_ref, z_ref, m_ref, s_ref):
    i = pl.program_id(0)
    zb = jnp.maximum(_dotT(fch_ref[...], w_ref[...]), 0.0)
    z_ref[...] = zb

    @pl.when(i == 0)
    def _():
        m_ref[...] = jnp.zeros_like(m_ref)
        s_ref[...] = jnp.zeros_like(s_ref)

    m_ref[...] += _dotT(zb, zb)
    s_ref[0:1, :] += jnp.sum(zb, axis=0, keepdims=True)


def _p3(fch, wa1):
    return pl.pallas_call(
        _p3_body,
        grid=(_NBLK,),
        in_specs=[pl.BlockSpec((16, _BN), lambda i: (0, i)),
                  pl.BlockSpec((16, 64), lambda i: (0, 0))],
        out_specs=[pl.BlockSpec((_BN, 64), lambda i: (i, 0)),
                   pl.BlockSpec((64, 64), lambda i: (0, 0)),
                   pl.BlockSpec((8, 64), lambda i: (0, 0))],
        out_shape=[jax.ShapeDtypeStruct((_NPAD, 64), jnp.float32),
                   jax.ShapeDtypeStruct((64, 64), jnp.float32),
                   jax.ShapeDtypeStruct((8, 64), jnp.float32)],
    )(fch, wa1)


def _p4_body(z_ref, w_ref, b_ref, o_ref, m_ref, s_ref):
    i = pl.program_id(0)
    zb = jnp.maximum(_dot(z_ref[...], w_ref[...]) + b_ref[0:1, :], 0.0)
    row = jax.lax.broadcasted_iota(jnp.int32, (_BN, 1), 0) + i * _BN
    zb = zb * (row < _N).astype(jnp.float32)
    o_ref[...] = zb

    @pl.when(i == 0)
    def _():
        m_ref[...] = jnp.zeros_like(m_ref)
        s_ref[...] = jnp.zeros_like(s_ref)

    m_ref[...] += _dotT(zb, zb)
    s_ref[0:1, :] += jnp.sum(zb, axis=0, keepdims=True)


def _p4(z1, w, b, din, dout):
    return pl.pallas_call(
        _p4_body,
        grid=(_NBLK,),
        in_specs=[pl.BlockSpec((_BN, din), lambda i: (i, 0)),
                  pl.BlockSpec((din, dout), lambda i: (0, 0)),
                  pl.BlockSpec((8, dout), lambda i: (0, 0))],
        out_specs=[pl.BlockSpec((_BN, dout), lambda i: (i, 0)),
                   pl.BlockSpec((dout, dout), lambda i: (0, 0)),
                   pl.BlockSpec((8, dout), lambda i: (0, 0))],
        out_shape=[jax.ShapeDtypeStruct((_NPAD, dout), jnp.float32),
                   jax.ShapeDtypeStruct((dout, dout), jnp.float32),
                   jax.ShapeDtypeStruct((8, dout), jnp.float32)],
    )(z1, w, jnp.broadcast_to(b[None, :], (8, dout)))


def _p5_body(z_ref, w3_ref, b3_ref, w4_ref, b4_ref, y0_ref, y1_ref, y2_ref, y3_ref):
    i = pl.program_id(0)
    z3 = jnp.maximum(_dot(z_ref[...], w3_ref[...], lax.Precision.DEFAULT)
                     + b3_ref[0:1, :], 0.0)
    y = _dot(z3, w4_ref[...], lax.Precision.DEFAULT) + b4_ref[0:1, :]
    row = jax.lax.broadcasted_iota(jnp.int32, (_BN, 1), 0) + i * _BN
    y = y * (row < _N).astype(jnp.float32)
    y0_ref[...] = y[:, 0:16]
    y1_ref[...] = y[:, 16:32]
    y2_ref[...] = y[:, 32:48]
    y3_ref[...] = y[:, 48:64]


def _p5(z2, w3, b3, w4, b4):
    return pl.pallas_call(
        _p5_body,
        grid=(_NBLK,),
        in_specs=[pl.BlockSpec((_BN, 128), lambda i: (i, 0)),
                  pl.BlockSpec((128, 256), lambda i: (0, 0)),
                  pl.BlockSpec((8, 256), lambda i: (0, 0)),
                  pl.BlockSpec((256, 64), lambda i: (0, 0)),
                  pl.BlockSpec((8, 64), lambda i: (0, 0))],
        out_specs=[pl.BlockSpec((_BN, 16), lambda i: (i, 0))] * 4,
        out_shape=[jax.ShapeDtypeStruct((_NPAD, 16), jnp.float32)] * 4,
    )(z2, w3, jnp.broadcast_to(b3[None, :], (8, 256)),
      w4, jnp.broadcast_to(b4[None, :], (8, 64)))


def _p6_body(t_ref, g_ref, o_ref):
    tT = lax.dot_general(jnp.eye(16, dtype=jnp.float32), t_ref[0, 0],
                         (((1,), (1,)), ((), ())), preferred_element_type=jnp.float32, precision=lax.Precision.HIGHEST)
    gT = lax.dot_general(jnp.eye(8, dtype=jnp.float32), g_ref[...],
                         (((1,), (1,)), ((), ())), preferred_element_type=jnp.float32, precision=lax.Precision.HIGHEST)
    cnt = jnp.maximum(gT[5:6, :], 1.0)
    o_ref[...] = (tT / cnt)[None]


def _p6(temp, grid6):
    nvb = _QH // _RB
    return pl.pallas_call(
        _p6_body,
        grid=(2, 2, 4, nvb),
        in_specs=[
            pl.BlockSpec((1, 1, _RB, 16),
                         lambda b, rh, cb, i: (b * 2 + rh, cb, i, 0)),
            pl.BlockSpec((_RB, 8),
                         lambda b, rh, cb, i: ((b * 2 + rh) * (_QH // _RB) + i, 0)),
        ],
        out_specs=pl.BlockSpec((1, 16, _RB),
                               lambda b, rh, cb, i: (b, cb, rh * nvb + i)),
        out_shape=jax.ShapeDtypeStruct((2, 64, 2 * _QH), jnp.float32),
    )(temp, grid6)


_NS = 16
_HALF = _NV // 2
_TPP = _NPAD // _NS
_NROW = _TPP // 128
_SDUM = 2048
_GDUM = 2048
_ZR = (_HALF + _SDUM) // _NS // 4


def _sca_body(keys_hbm, vals_hbm, zeros_hbm, grid6_hbm, gath_hbm,
              sbuf, kbuf, ibuf, vbuf, gbuf):
    c = lax.axis_index("c")
    s = lax.axis_index("s")
    base = s * _TPP
    cbase = c * _HALF
    lane = lax.broadcasted_iota(jnp.int32, (16,), 0)

    pltpu.sync_copy(zeros_hbm, sbuf.at[pl.ds(s * (4 * _ZR), 4 * _ZR)])

    pltpu.sync_copy(keys_hbm.at[pl.ds(base, _TPP)], kbuf)

    @pl.loop(0, _NROW)
    def _prep(g):
        for jj in range(8):
            v = g * 128 + jj * 16
            k = kbuf[pl.ds(v, 16)]
            local = k - cbase
            owned = (local >= 0) & (local < _HALF)
            sdum = _HALF + s * 128 + jj * 16 + lane
            ibuf[g, pl.ds(jj * 16, 16)] = jnp.where(owned, local, sdum)

    plsc.subcore_barrier()

    @pl.loop(0, _NROW)
    def _scat(g):
        pltpu.sync_copy(vals_hbm.at[pl.ds(base + g * 128, 128)], vbuf)
        pltpu.sync_copy(vbuf, sbuf.at[ibuf.at[g]], add=True)

    plsc.subcore_barrier()

    @pl.loop(0, _NROW)
    def _gath(g):
        pltpu.sync_copy(sbuf.at[ibuf.at[g]], gbuf)
        pltpu.sync_copy(gbuf, gath_hbm.at[c, pl.ds(base + g * 128, 128)])

    rows = _HALF // _NS
    pltpu.sync_copy(sbuf.at[pl.ds(s * rows, rows)],
                    grid6_hbm.at[pl.ds(cbase + s * rows, rows)])


@functools.lru_cache(maxsize=None)
def _sca():
    return pl.kernel(
        _sca_body,
        out_type=[jax.ShapeDtypeStruct((_NV, 8), jnp.float32),
                  jax.ShapeDtypeStruct((2, _NPAD, 8), jnp.float32)],
        mesh=plsc.VectorSubcoreMesh(core_axis_name="c", subcore_axis_name="s",
                                    num_cores=2, num_subcores=_NS),
        compiler_params=pltpu.CompilerParams(use_tc_tiling_on_sc=False),
        scratch_types=[
            pltpu.VMEM_SHARED((_HALF + _SDUM, 8), jnp.float32),
            pltpu.VMEM((_TPP,), jnp.int32),
            pltpu.VMEM((_NROW, 128), jnp.int32),
            pltpu.VMEM((128, 8), jnp.float32),
            pltpu.VMEM((128, 8), jnp.float32),
        ],
    )


_EROWS = _QH + _SDUM
_EZR = _EROWS // _NS


def _sce_body(keys_hbm, y0_hbm, y1_hbm, y2_hbm, y3_hbm, zeros_hbm, temp_hbm,
              ebuf, kbuf, ibuf, vbuf):
    c = lax.axis_index("c")
    s = lax.axis_index("s")
    base = s * _TPP
    lane = lax.broadcasted_iota(jnp.int32, (16,), 0)
    xrows = _QH // _NS

    pltpu.sync_copy(keys_hbm.at[pl.ds(base, _TPP)], kbuf)
    y_hbms = (y0_hbm, y1_hbm, y2_hbm, y3_hbm)

    for h in range(2):
        rc = c * 2 + h
        rbase = rc * _QH

        @pl.loop(0, _NROW)
        def _prep(g):
            for jj in range(8):
                v = g * 128 + jj * 16
                k = kbuf[pl.ds(v, 16)]
                local = k - rbase
                owned = (local >= 0) & (local < _QH)
                sdum = _QH + s * 128 + jj * 16 + lane
                ibuf[g, pl.ds(jj * 16, 16)] = jnp.where(owned, local, sdum)

        for cb in range(4):
            plsc.subcore_barrier()
            pltpu.sync_copy(zeros_hbm, ebuf.at[pl.ds(s * _EZR, _EZR)])
            plsc.subcore_barrier()
            y_hbm = y_hbms[cb]

            @pl.loop(0, _NROW)
            def _scat(g):
                pltpu.sync_copy(y_hbm.at[pl.ds(base + g * 128, 128)], vbuf)
                pltpu.sync_copy(vbuf, ebuf.at[ibuf.at[g]], add=True)

            plsc.subcore_barrier()
            pltpu.sync_copy(ebuf.at[pl.ds(s * xrows, xrows)],
                            temp_hbm.at[rc, cb, pl.ds(s * xrows, xrows)])


@functools.lru_cache(maxsize=None)
def _sce():
    return pl.kernel(
        _sce_body,
        out_type=jax.ShapeDtypeStruct((4, 4, _QH, 16), jnp.float32),
        mesh=plsc.VectorSubcoreMesh(core_axis_name="c", subcore_axis_name="s",
                                    num_cores=2, num_subcores=_NS),
        compiler_params=pltpu.CompilerParams(use_tc_tiling_on_sc=False),
        scratch_types=[
            pltpu.VMEM_SHARED((_EROWS, 16), jnp.float32),
            pltpu.VMEM((_TPP,), jnp.int32),
            pltpu.VMEM((_NROW, 128), jnp.int32),
            pltpu.VMEM((128, 16), jnp.float32),
        ],
    )


def _fold1(G, bn0_g, bn0_b, w1, b1, bn1_g, bn1_b):
    n = G[14, 14]
    sf = G[14, :14]
    f2 = G[:14, :14]
    mu = sf / n
    var0 = jnp.maximum(jnp.diag(f2) / n - mu * mu, 0.0)
    a0 = bn0_g / jnp.sqrt(var0 + _EPS)
    c0 = bn0_b - mu * a0
    A1 = a0[:, None] * w1
    d1 = c0 @ w1 + b1
    mu1 = mu @ A1 + d1
    Cf = f2 / n - mu[:, None] * mu[None, :]
    var1 = jnp.maximum(jnp.sum(A1 * (Cf @ A1), axis=0), 0.0)
    s1 = bn1_g / jnp.sqrt(var1 + _EPS)
    w1e = A1 * s1[None, :]
    b1e = (d1 - mu1) * s1 + bn1_b
    wa = jnp.zeros((16, 64), jnp.float32)
    wa = wa.at[:14, :].set(w1e).at[14, :].set(b1e)
    return wa


def _fold_mid(M, S, w, b, bng, bnb):
    n = jnp.float32(_N)
    muz = S[0] / n
    Cz = M / n - muz[:, None] * muz[None, :]
    muy = muz @ w + b
    var = jnp.maximum(jnp.sum(w * (Cz @ w), axis=0), 0.0)
    t = bng / jnp.sqrt(var + _EPS)
    return w * t[None, :], (b - muy) * t + bnb


def kernel(points, batch_size, bn0_g, bn0_b, lin1_w, lin1_b, bn1_g, bn1_b,
           lin2_w, lin2_b, bn2_g, bn2_b, lin3_w, lin3_b, bn3_g, bn3_b,
           lin4_w, lin4_b):
    f32 = jnp.float32
    pts_t = jnp.zeros((8, _NPAD), f32).at[:5, :_N].set(points.T)
    valsA = _p1(pts_t)
    rho0 = jnp.sqrt(points[:, 1] ** 2 + points[:, 2] ** 2)
    phi0 = jnp.arctan2(points[:, 2], points[:, 1])
    vr0 = jnp.clip(jnp.floor((rho0 - _PCR[0]) / _VOX[0]).astype(jnp.int32), 0, _GRID[0] - 1)
    vp0 = jnp.clip(jnp.floor((phi0 - _PCR[1]) / _VOX[1]).astype(jnp.int32), 0, _GRID[1] - 1)
    b0 = points[:, 0].astype(jnp.int32)
    key0 = (b0 * _GRID[0] + vr0) * _GRID[1] + vp0
    keys = jnp.full((_NPAD,), _SENT, jnp.int32).at[:_N].set(key0)

    zeros8 = jnp.zeros((4 * _ZR, 8), f32)
    grid6, gathered = _sca()(keys, valsA, zeros8)

    fch, G = _p2(pts_t, gathered, keys.reshape(_NBLK, 1, _BN))
    wa1 = _fold1(G, bn0_g, bn0_b, lin1_w, lin1_b, bn1_g, bn1_b)
    z1, M1, S1 = _p3(fch, wa1)
    w2e, b2e = _fold_mid(M1, S1, lin2_w, lin2_b, bn2_g, bn2_b)
    z2, M2, S2 = _p4(z1, w2e, b2e, 64, 128)
    w3e, b3e = _fold_mid(M2, S2, lin3_w, lin3_b, bn3_g, bn3_b)
    y0, y1, y2, y3 = _p5(z2, w3e, b3e, lin4_w, lin4_b)

    zeros16 = jnp.zeros((_EZR, 16), f32)
    temp = _sce()(keys, y0, y1, y2, y3, zeros16)

    out = _p6(temp, grid6)
    return out.reshape(2, 64, _GRID[0], _GRID[1])


# speedup vs baseline: 1.1110x; 1.1110x over previous
import functools

import jax
import jax.numpy as jnp
import numpy as np
from jax import lax
from jax.experimental import pallas as pl
from jax.experimental.pallas import tpu as pltpu
from jax.experimental.pallas import tpu_sc as plsc

_GRID = (480, 360, 32)
_PCR = np.array([0.0, -np.pi, -4.0, 50.0, np.pi, 2.0], dtype=np.float32)
_VOX = np.array([(_PCR[3] - _PCR[0]) / _GRID[0], (_PCR[4] - _PCR[1]) / _GRID[1],
                 (_PCR[5] - _PCR[2]) / _GRID[2]], dtype=np.float32)
_NV = 2 * _GRID[0] * _GRID[1]
_N = 200000
_NPAD = 204800
_BN = 1024
_NBLK = _NPAD // _BN
_SENT = 1 << 29
_EPS = 1e-5
_RB = 3456
_QH = 86400


def _dotT(a, b):
    return lax.dot_general(a, b, (((0,), (0,)), ((), ())),
                           preferred_element_type=jnp.float32, precision=lax.Precision.HIGHEST)


def _dot(a, b, prec=lax.Precision.HIGHEST):
    return lax.dot_general(a, b, (((1,), (0,)), ((), ())),
                           preferred_element_type=jnp.float32, precision=prec)


def _p1_body(pts_ref, vals_ref):
    i = pl.program_id(0)
    p = pts_ref[...]
    x = p[1:2, :]
    y = p[2:3, :]
    z = p[3:4, :]
    rho = jnp.sqrt(x * x + y * y)
    phi = jnp.arctan2(y, x)
    col = jax.lax.broadcasted_iota(jnp.int32, (1, _BN), 1) + i * _BN
    valid = col < _N
    vmask = valid.astype(jnp.float32)
    ones = jnp.ones((1, _BN), jnp.float32)
    zeros = jnp.zeros((1, _BN), jnp.float32)
    vch = jnp.concatenate([rho, phi, z, x, y, ones, zeros, zeros], axis=0) * vmask
    vals_ref[...] = _dotT(vch, jnp.eye(8, dtype=jnp.float32))


def _p1(pts_t):
    return pl.pallas_call(
        _p1_body,
        grid=(_NBLK,),
        in_specs=[pl.BlockSpec((8, _BN), lambda i: (0, i))],
        out_specs=pl.BlockSpec((_BN, 8), lambda i: (i, 0)),
        out_shape=jax.ShapeDtypeStruct((_NPAD, 8), jnp.float32),
    )(pts_t)


def _p2_body(pts_ref, g_ref, k_ref, fch_ref, gram_ref):
    i = pl.program_id(0)
    p = pts_ref[...]
    x = p[1:2, :]
    y = p[2:3, :]
    z = p[3:4, :]
    inten = p[4:5, :]
    rho = jnp.sqrt(x * x + y * y)
    phi = jnp.arctan2(y, x)
    k = k_ref[0]
    vp = k % _GRID[1]
    vr = (k // _GRID[1]) % _GRID[0]
    vz = jnp.clip(jnp.floor((z - _PCR[2]) / _VOX[2]).astype(jnp.int32), 0, _GRID[2] - 1)
    cr = (vr.astype(jnp.float32) + 0.5) * _VOX[0] + _PCR[0]
    cp = (vp.astype(jnp.float32) + 0.5) * _VOX[1] + _PCR[1]
    cz = (vz.astype(jnp.float32) + 0.5) * _VOX[2] + _PCR[2]
    ey8 = jnp.eye(8, dtype=jnp.float32)
    gT0 = lax.dot_general(ey8, g_ref[0], (((1,), (1,)), ((), ())),
                          preferred_element_type=jnp.float32, precision=lax.Precision.HIGHEST)
    gT1 = lax.dot_general(ey8, g_ref[1], (((1,), (1,)), ((), ())),
                          preferred_element_type=jnp.float32, precision=lax.Precision.HIGHEST)
    gT = jnp.where(k < _HALF, gT0, gT1)
    cnt = jnp.maximum(gT[5:6, :], 1.0)
    mean5 = gT[0:5, :] / cnt
    pc5 = jnp.concatenate([rho, phi, z, x, y], axis=0)
    nor = pc5 - mean5
    c2p = jnp.concatenate([rho - cr, phi - cp, z - cz], axis=0)
    col = jax.lax.broadcasted_iota(jnp.int32, (1, _BN), 1) + i * _BN
    vmask = (col < _N).astype(jnp.float32)
    ones = jnp.ones((1, _BN), jnp.float32)
    zeros = jnp.zeros((1, _BN), jnp.float32)
    fch = jnp.concatenate([pc5, inten, nor, c2p, ones, zeros], axis=0) * vmask
    fch_ref[...] = fch

    @pl.when(i == 0)
    def _():
        gram_ref[...] = jnp.zeros_like(gram_ref)

    gram_ref[...] += lax.dot_general(fch, fch, (((1,), (1,)), ((), ())),
                                     preferred_element_type=jnp.float32, precision=lax.Precision.HIGHEST)


def _p2(pts_t, gathered, keys3):
    return pl.pallas_call(
        _p2_body,
        grid=(_NBLK,),
        in_specs=[pl.BlockSpec((8, _BN), lambda i: (0, i)),
                  pl.BlockSpec((2, _BN, 8), lambda i: (0, i, 0)),
                  pl.BlockSpec((1, 1, _BN), lambda i: (i, 0, 0))],
        out_specs=[pl.BlockSpec((16, _BN), lambda i: (0, i)),
                   pl.BlockSpec((16, 16), lambda i: (0, 0))],
        out_shape=[jax.ShapeDtypeStruct((16, _NPAD), jnp.float32),
                   jax.ShapeDtypeStruct((16, 16), jnp.float32)],
    )(pts_t, gathered, keys3)


def _p3_body(fch_ref, w_ref, z_ref, m_ref, s_ref):
    i = pl.program_id(0)
    zb = jnp.maximum(_dotT(fch_ref[...], w_ref[...]), 0.0)
    z_ref[...] = zb

    @pl.when(i == 0)
    def _():
        m_ref[...] = jnp.zeros_like(m_ref)
        s_ref[...] = jnp.zeros_like(s_ref)

    m_ref[...] += _dotT(zb, zb)
    s_ref[0:1, :] += jnp.sum(zb, axis=0, keepdims=True)


def _p3(fch, wa1):
    return pl.pallas_call(
        _p3_body,
        grid=(_NBLK,),
        in_specs=[pl.BlockSpec((16, _BN), lambda i: (0, i)),
                  pl.BlockSpec((16, 64), lambda i: (0, 0))],
        out_specs=[pl.BlockSpec((_BN, 64), lambda i: (i, 0)),
                   pl.BlockSpec((64, 64), lambda i: (0, 0)),
                   pl.BlockSpec((8, 64), lambda i: (0, 0))],
        out_shape=[jax.ShapeDtypeStruct((_NPAD, 64), jnp.float32),
                   jax.ShapeDtypeStruct((64, 64), jnp.float32),
                   jax.ShapeDtypeStruct((8, 64), jnp.float32)],
    )(fch, wa1)


def _p4_body(z_ref, w_ref, b_ref, o_ref, m_ref, s_ref):
    i = pl.program_id(0)
    zb = jnp.maximum(_dot(z_ref[...], w_ref[...]) + b_ref[0:1, :], 0.0)
    row = jax.lax.broadcasted_iota(jnp.int32, (_BN, 1), 0) + i * _BN
    zb = zb * (row < _N).astype(jnp.float32)
    o_ref[...] = zb

    @pl.when(i == 0)
    def _():
        m_ref[...] = jnp.zeros_like(m_ref)
        s_ref[...] = jnp.zeros_like(s_ref)

    m_ref[...] += _dotT(zb, zb)
    s_ref[0:1, :] += jnp.sum(zb, axis=0, keepdims=True)


def _p4(z1, w, b, din, dout):
    return pl.pallas_call(
        _p4_body,
        grid=(_NBLK,),
        in_specs=[pl.BlockSpec((_BN, din), lambda i: (i, 0)),
                  pl.BlockSpec((din, dout), lambda i: (0, 0)),
                  pl.BlockSpec((8, dout), lambda i: (0, 0))],
        out_specs=[pl.BlockSpec((_BN, dout), lambda i: (i, 0)),
                   pl.BlockSpec((dout, dout), lambda i: (0, 0)),
                   pl.BlockSpec((8, dout), lambda i: (0, 0))],
        out_shape=[jax.ShapeDtypeStruct((_NPAD, dout), jnp.float32),
                   jax.ShapeDtypeStruct((dout, dout), jnp.float32),
                   jax.ShapeDtypeStruct((8, dout), jnp.float32)],
    )(z1, w, jnp.broadcast_to(b[None, :], (8, dout)))


def _p5_body(z_ref, w3_ref, b3_ref, w4_ref, b4_ref, y0_ref, y1_ref, y2_ref, y3_ref):
    i = pl.program_id(0)
    z3 = jnp.maximum(_dot(z_ref[...], w3_ref[...], lax.Precision.DEFAULT)
                     + b3_ref[0:1, :], 0.0)
    y = _dot(z3, w4_ref[...], lax.Precision.DEFAULT) + b4_ref[0:1, :]
    row = jax.lax.broadcasted_iota(jnp.int32, (_BN, 1), 0) + i * _BN
    y = y * (row < _N).astype(jnp.float32)
    y0_ref[...] = y[:, 0:16]
    y1_ref[...] = y[:, 16:32]
    y2_ref[...] = y[:, 32:48]
    y3_ref[...] = y[:, 48:64]


def _p5(z2, w3, b3, w4, b4):
    return pl.pallas_call(
        _p5_body,
        grid=(_NBLK,),
        in_specs=[pl.BlockSpec((_BN, 128), lambda i: (i, 0)),
                  pl.BlockSpec((128, 256), lambda i: (0, 0)),
                  pl.BlockSpec((8, 256), lambda i: (0, 0)),
                  pl.BlockSpec((256, 64), lambda i: (0, 0)),
                  pl.BlockSpec((8, 64), lambda i: (0, 0))],
        out_specs=[pl.BlockSpec((_BN, 16), lambda i: (i, 0))] * 4,
        out_shape=[jax.ShapeDtypeStruct((_NPAD, 16), jnp.float32)] * 4,
    )(z2, w3, jnp.broadcast_to(b3[None, :], (8, 256)),
      w4, jnp.broadcast_to(b4[None, :], (8, 64)))


def _p6_body(t_ref, g_ref, o_ref):
    tT = lax.dot_general(jnp.eye(16, dtype=jnp.float32), t_ref[0, 0],
                         (((1,), (1,)), ((), ())), preferred_element_type=jnp.float32, precision=lax.Precision.HIGHEST)
    gT = lax.dot_general(jnp.eye(8, dtype=jnp.float32), g_ref[...],
                         (((1,), (1,)), ((), ())), preferred_element_type=jnp.float32, precision=lax.Precision.HIGHEST)
    cnt = jnp.maximum(gT[5:6, :], 1.0)
    o_ref[...] = (tT / cnt)[None]


def _p6(temp, grid6):
    nvb = _QH // _RB
    return pl.pallas_call(
        _p6_body,
        grid=(2, 2, 4, nvb),
        in_specs=[
            pl.BlockSpec((1, 1, _RB, 16),
                         lambda b, rh, cb, i: (b * 2 + rh, cb, i, 0)),
            pl.BlockSpec((_RB, 8),
                         lambda b, rh, cb, i: ((b * 2 + rh) * (_QH // _RB) + i, 0)),
        ],
        out_specs=pl.BlockSpec((1, 16, _RB),
                               lambda b, rh, cb, i: (b, cb, rh * nvb + i)),
        out_shape=jax.ShapeDtypeStruct((2, 64, 2 * _QH), jnp.float32),
    )(temp, grid6)


_NS = 16
_HALF = _NV // 2
_TPP = _NPAD // _NS
_NROW = _TPP // 128
_SDUM = 2048
_GDUM = 2048
_ZR = (_HALF + _SDUM) // _NS // 4


def _sca_body(keys_hbm, vals_hbm, zeros_hbm, grid6_hbm, gath_hbm,
              sbuf, kbuf, ibuf, vbuf, gbuf):
    c = lax.axis_index("c")
    s = lax.axis_index("s")
    base = s * _TPP
    cbase = c * _HALF
    lane = lax.broadcasted_iota(jnp.int32, (16,), 0)

    pltpu.sync_copy(zeros_hbm, sbuf.at[pl.ds(s * (4 * _ZR), 4 * _ZR)])

    pltpu.sync_copy(keys_hbm.at[pl.ds(base, _TPP)], kbuf)

    @pl.loop(0, _NROW)
    def _prep(g):
        for jj in range(8):
            v = g * 128 + jj * 16
            k = kbuf[pl.ds(v, 16)]
            local = k - cbase
            owned = (local >= 0) & (local < _HALF)
            sdum = _HALF + s * 128 + jj * 16 + lane
            ibuf[g, pl.ds(jj * 16, 16)] = jnp.where(owned, local, sdum)

    plsc.subcore_barrier()

    @pl.loop(0, _NROW)
    def _scat(g):
        pltpu.sync_copy(vals_hbm.at[pl.ds(base + g * 128, 128)], vbuf)
        pltpu.sync_copy(vbuf, sbuf.at[ibuf.at[g]], add=True)

    plsc.subcore_barrier()

    @pl.loop(0, _NROW)
    def _gath(g):
        pltpu.sync_copy(sbuf.at[ibuf.at[g]], gbuf)
        pltpu.sync_copy(gbuf, gath_hbm.at[c, pl.ds(base + g * 128, 128)])

    rows = _HALF // _NS
    pltpu.sync_copy(sbuf.at[pl.ds(s * rows, rows)],
                    grid6_hbm.at[pl.ds(cbase + s * rows, rows)])


@functools.lru_cache(maxsize=None)
def _sca():
    return pl.kernel(
        _sca_body,
        out_type=[jax.ShapeDtypeStruct((_NV, 8), jnp.float32),
                  jax.ShapeDtypeStruct((2, _NPAD, 8), jnp.float32)],
        mesh=plsc.VectorSubcoreMesh(core_axis_name="c", subcore_axis_name="s",
                                    num_cores=2, num_subcores=_NS),
        compiler_params=pltpu.CompilerParams(use_tc_tiling_on_sc=False),
        scratch_types=[
            pltpu.VMEM_SHARED((_HALF + _SDUM, 8), jnp.float32),
            pltpu.VMEM((_TPP,), jnp.int32),
            pltpu.VMEM((_NROW, 128), jnp.int32),
            pltpu.VMEM((128, 8), jnp.float32),
            pltpu.VMEM((128, 8), jnp.float32),
        ],
    )


_EROWS = _QH + 512
_EZR = _EROWS // _NS


def _sce_body(keys_hbm, y0_hbm, y1_hbm, y2_hbm, y3_hbm, zeros_hbm, temp_hbm,
              ebuf, kbuf, ibuf, vbuf, sem):
    c = lax.axis_index("c")
    s = lax.axis_index("s")
    base = s * _TPP
    lane = lax.broadcasted_iota(jnp.int32, (16,), 0)
    xrows = _QH // _NS
    y_hbms = (y0_hbm, y1_hbm, y2_hbm, y3_hbm)

    for h in range(2):
        rc = c * 2 + h
        rbase = rc * _QH

        for part in range(2):
            pltpu.sync_copy(keys_hbm.at[pl.ds(base + part * 6400, 6400)], kbuf)

            @pl.loop(0, _NROW // 2)
            def _prep(g):
                for jj in range(8):
                    v = g * 128 + jj * 16
                    k = kbuf[pl.ds(v, 16)]
                    local = k - rbase
                    owned = (local >= 0) & (local < _QH)
                    sdum = _QH + s * 32 + (jj % 2) * 16 + lane
                    ibuf[part * (_NROW // 2) + g, pl.ds(jj * 16, 16)] = \
                        jnp.where(owned, local, sdum)

        for cb in range(4):
            plsc.subcore_barrier()
            pltpu.sync_copy(zeros_hbm, ebuf.at[pl.ds(s * _EZR, _EZR)])
            plsc.subcore_barrier()
            y_hbm = y_hbms[cb]

            for q in range(10):
                pltpu.sync_copy(y_hbm.at[pl.ds(base + q * 1280, 1280)], vbuf)

                @pl.loop(0, 10)
                def _fire(g):
                    pltpu.async_copy(vbuf.at[pl.ds(g * 128, 128)],
                                     ebuf.at[ibuf.at[q * 10 + g]], sem, add=True)

                @pl.loop(0, 10)
                def _drain(g):
                    pltpu.make_async_copy(zeros_hbm.at[pl.ds(0, 128)],
                                          vbuf.at[pl.ds(0, 128)], sem).wait()

            plsc.subcore_barrier()
            pltpu.sync_copy(ebuf.at[pl.ds(s * xrows, xrows)],
                            temp_hbm.at[rc, cb, pl.ds(s * xrows, xrows)])


@functools.lru_cache(maxsize=None)
def _sce():
    return pl.kernel(
        _sce_body,
        out_type=jax.ShapeDtypeStruct((4, 4, _QH, 16), jnp.float32),
        mesh=plsc.VectorSubcoreMesh(core_axis_name="c", subcore_axis_name="s",
                                    num_cores=2, num_subcores=_NS),
        compiler_params=pltpu.CompilerParams(use_tc_tiling_on_sc=False),
        scratch_types=[
            pltpu.VMEM_SHARED((_EROWS, 16), jnp.float32),
            pltpu.VMEM((6400,), jnp.int32),
            pltpu.VMEM((_NROW, 128), jnp.int32),
            pltpu.VMEM((1280, 16), jnp.float32),
            pltpu.SemaphoreType.DMA,
        ],
    )


def _fold1(G, bn0_g, bn0_b, w1, b1, bn1_g, bn1_b):
    n = G[14, 14]
    sf = G[14, :14]
    f2 = G[:14, :14]
    mu = sf / n
    var0 = jnp.maximum(jnp.diag(f2) / n - mu * mu, 0.0)
    a0 = bn0_g / jnp.sqrt(var0 + _EPS)
    c0 = bn0_b - mu * a0
    A1 = a0[:, None] * w1
    d1 = c0 @ w1 + b1
    mu1 = mu @ A1 + d1
    Cf = f2 / n - mu[:, None] * mu[None, :]
    var1 = jnp.maximum(jnp.sum(A1 * (Cf @ A1), axis=0), 0.0)
    s1 = bn1_g / jnp.sqrt(var1 + _EPS)
    w1e = A1 * s1[None, :]
    b1e = (d1 - mu1) * s1 + bn1_b
    wa = jnp.zeros((16, 64), jnp.float32)
    wa = wa.at[:14, :].set(w1e).at[14, :].set(b1e)
    return wa


def _fold_mid(M, S, w, b, bng, bnb):
    n = jnp.float32(_N)
    muz = S[0] / n
    Cz = M / n - muz[:, None] * muz[None, :]
    muy = muz @ w + b
    var = jnp.maximum(jnp.sum(w * (Cz @ w), axis=0), 0.0)
    t = bng / jnp.sqrt(var + _EPS)
    return w * t[None, :], (b - muy) * t + bnb


def kernel(points, batch_size, bn0_g, bn0_b, lin1_w, lin1_b, bn1_g, bn1_b,
           lin2_w, lin2_b, bn2_g, bn2_b, lin3_w, lin3_b, bn3_g, bn3_b,
           lin4_w, lin4_b):
    f32 = jnp.float32
    pts_t = jnp.zeros((8, _NPAD), f32).at[:5, :_N].set(points.T)
    valsA = _p1(pts_t)
    rho0 = jnp.sqrt(points[:, 1] ** 2 + points[:, 2] ** 2)
    phi0 = jnp.arctan2(points[:, 2], points[:, 1])
    vr0 = jnp.clip(jnp.floor((rho0 - _PCR[0]) / _VOX[0]).astype(jnp.int32), 0, _GRID[0] - 1)
    vp0 = jnp.clip(jnp.floor((phi0 - _PCR[1]) / _VOX[1]).astype(jnp.int32), 0, _GRID[1] - 1)
    b0 = points[:, 0].astype(jnp.int32)
    key0 = (b0 * _GRID[0] + vr0) * _GRID[1] + vp0
    keys = jnp.full((_NPAD,), _SENT, jnp.int32).at[:_N].set(key0)

    zeros8 = jnp.zeros((4 * _ZR, 8), f32)
    grid6, gathered = _sca()(keys, valsA, zeros8)

    fch, G = _p2(pts_t, gathered, keys.reshape(_NBLK, 1, _BN))
    wa1 = _fold1(G, bn0_g, bn0_b, lin1_w, lin1_b, bn1_g, bn1_b)
    z1, M1, S1 = _p3(fch, wa1)
    w2e, b2e = _fold_mid(M1, S1, lin2_w, lin2_b, bn2_g, bn2_b)
    z2, M2, S2 = _p4(z1, w2e, b2e, 64, 128)
    w3e, b3e = _fold_mid(M2, S2, lin3_w, lin3_b, bn3_g, bn3_b)
    y0, y1, y2, y3 = _p5(z2, w3e, b3e, lin4_w, lin4_b)

    zeros16 = jnp.zeros((_EZR, 16), f32)
    temp = _sce()(keys, y0, y1, y2, y3, zeros16)

    out = _p6(temp, grid6)
    return out.reshape(2, 64, _GRID[0], _GRID[1])


# speedup vs baseline: 1.1904x; 1.0715x over previous
import functools

import jax
import jax.numpy as jnp
import numpy as np
from jax import lax
from jax.experimental import pallas as pl
from jax.experimental.pallas import tpu as pltpu
from jax.experimental.pallas import tpu_sc as plsc

_GRID = (480, 360, 32)
_PCR = np.array([0.0, -np.pi, -4.0, 50.0, np.pi, 2.0], dtype=np.float32)
_VOX = np.array([(_PCR[3] - _PCR[0]) / _GRID[0], (_PCR[4] - _PCR[1]) / _GRID[1],
                 (_PCR[5] - _PCR[2]) / _GRID[2]], dtype=np.float32)
_NV = 2 * _GRID[0] * _GRID[1]
_N = 200000
_NPAD = 204800
_BN = 1024
_NBLK = _NPAD // _BN
_SENT = 1 << 29
_EPS = 1e-5
_RB = 3456
_QH = 86400


def _dotT(a, b):
    return lax.dot_general(a, b, (((0,), (0,)), ((), ())),
                           preferred_element_type=jnp.float32, precision=lax.Precision.HIGHEST)


def _dot(a, b, prec=lax.Precision.HIGHEST):
    return lax.dot_general(a, b, (((1,), (0,)), ((), ())),
                           preferred_element_type=jnp.float32, precision=prec)


def _p1_body(pts_ref, vals_ref):
    i = pl.program_id(0)
    p = pts_ref[...]
    x = p[1:2, :]
    y = p[2:3, :]
    z = p[3:4, :]
    rho = jnp.sqrt(x * x + y * y)
    phi = jnp.arctan2(y, x)
    col = jax.lax.broadcasted_iota(jnp.int32, (1, _BN), 1) + i * _BN
    valid = col < _N
    vmask = valid.astype(jnp.float32)
    ones = jnp.ones((1, _BN), jnp.float32)
    zeros = jnp.zeros((1, _BN), jnp.float32)
    vch = jnp.concatenate([rho, phi, z, x, y, ones, zeros, zeros], axis=0) * vmask
    vals_ref[...] = _dotT(vch, jnp.eye(8, dtype=jnp.float32))


def _p1(pts_t):
    return pl.pallas_call(
        _p1_body,
        grid=(_NBLK,),
        in_specs=[pl.BlockSpec((8, _BN), lambda i: (0, i))],
        out_specs=pl.BlockSpec((_BN, 8), lambda i: (i, 0)),
        out_shape=jax.ShapeDtypeStruct((_NPAD, 8), jnp.float32),
    )(pts_t)


def _p2_body(pts_ref, g_ref, k_ref, fch_ref, gram_ref):
    i = pl.program_id(0)
    p = pts_ref[...]
    x = p[1:2, :]
    y = p[2:3, :]
    z = p[3:4, :]
    inten = p[4:5, :]
    rho = jnp.sqrt(x * x + y * y)
    phi = jnp.arctan2(y, x)
    k = k_ref[0]
    vp = k % _GRID[1]
    vr = (k // _GRID[1]) % _GRID[0]
    vz = jnp.clip(jnp.floor((z - _PCR[2]) / _VOX[2]).astype(jnp.int32), 0, _GRID[2] - 1)
    cr = (vr.astype(jnp.float32) + 0.5) * _VOX[0] + _PCR[0]
    cp = (vp.astype(jnp.float32) + 0.5) * _VOX[1] + _PCR[1]
    cz = (vz.astype(jnp.float32) + 0.5) * _VOX[2] + _PCR[2]
    ey8 = jnp.eye(8, dtype=jnp.float32)
    gT0 = lax.dot_general(ey8, g_ref[0], (((1,), (1,)), ((), ())),
                          preferred_element_type=jnp.float32, precision=lax.Precision.HIGHEST)
    gT1 = lax.dot_general(ey8, g_ref[1], (((1,), (1,)), ((), ())),
                          preferred_element_type=jnp.float32, precision=lax.Precision.HIGHEST)
    gT = jnp.where(k < _HALF, gT0, gT1)
    cnt = jnp.maximum(gT[5:6, :], 1.0)
    mean5 = gT[0:5, :] / cnt
    pc5 = jnp.concatenate([rho, phi, z, x, y], axis=0)
    nor = pc5 - mean5
    c2p = jnp.concatenate([rho - cr, phi - cp, z - cz], axis=0)
    col = jax.lax.broadcasted_iota(jnp.int32, (1, _BN), 1) + i * _BN
    vmask = (col < _N).astype(jnp.float32)
    ones = jnp.ones((1, _BN), jnp.float32)
    zeros = jnp.zeros((1, _BN), jnp.float32)
    fch = jnp.concatenate([pc5, inten, nor, c2p, ones, zeros], axis=0) * vmask
    fch_ref[...] = fch

    @pl.when(i == 0)
    def _():
        gram_ref[...] = jnp.zeros_like(gram_ref)

    gram_ref[...] += lax.dot_general(fch, fch, (((1,), (1,)), ((), ())),
                                     preferred_element_type=jnp.float32, precision=lax.Precision.HIGHEST)


def _p2(pts_t, gathered, keys3):
    return pl.pallas_call(
        _p2_body,
        grid=(_NBLK,),
        in_specs=[pl.BlockSpec((8, _BN), lambda i: (0, i)),
                  pl.BlockSpec((2, _BN, 8), lambda i: (0, i, 0)),
                  pl.BlockSpec((1, 1, _BN), lambda i: (i, 0, 0))],
        out_specs=[pl.BlockSpec((16, _BN), lambda i: (0, i)),
                   pl.BlockSpec((16, 16), lambda i: (0, 0))],
        out_shape=[jax.ShapeDtypeStruct((16, _NPAD), jnp.float32),
                   jax.ShapeDtypeStruct((16, 16), jnp.float32)],
    )(pts_t, gathered, keys3)


def _p3_body(fch_ref, w_ref, z_ref, m_ref, s_ref):
    i = pl.program_id(0)
    zb = jnp.maximum(_dotT(fch_ref[...], w_ref[...]), 0.0)
    z_ref[...] = zb

    @pl.when(i == 0)
    def _():
        m_ref[...] = jnp.zeros_like(m_ref)
        s_ref[...] = jnp.zeros_like(s_ref)

    m_ref[...] += lax.dot_general(zb, zb, (((0,), (0,)), ((), ())),
                                  preferred_element_type=jnp.float32,
                                  precision=lax.Precision.DEFAULT)
    s_ref[0:1, :] += jnp.sum(zb, axis=0, keepdims=True)


def _p3(fch, wa1):
    return pl.pallas_call(
        _p3_body,
        grid=(_NBLK,),
        in_specs=[pl.BlockSpec((16, _BN), lambda i: (0, i)),
                  pl.BlockSpec((16, 64), lambda i: (0, 0))],
        out_specs=[pl.BlockSpec((_BN, 64), lambda i: (i, 0)),
                   pl.BlockSpec((64, 64), lambda i: (0, 0)),
                   pl.BlockSpec((8, 64), lambda i: (0, 0))],
        out_shape=[jax.ShapeDtypeStruct((_NPAD, 64), jnp.float32),
                   jax.ShapeDtypeStruct((64, 64), jnp.float32),
                   jax.ShapeDtypeStruct((8, 64), jnp.float32)],
    )(fch, wa1)


def _p4_body(z_ref, w_ref, b_ref, o_ref, m_ref, s_ref):
    i = pl.program_id(0)
    zb = jnp.maximum(_dot(z_ref[...], w_ref[...]) + b_ref[0:1, :], 0.0)
    row = jax.lax.broadcasted_iota(jnp.int32, (_BN, 1), 0) + i * _BN
    zb = zb * (row < _N).astype(jnp.float32)
    o_ref[...] = zb

    @pl.when(i == 0)
    def _():
        m_ref[...] = jnp.zeros_like(m_ref)
        s_ref[...] = jnp.zeros_like(s_ref)

    m_ref[...] += lax.dot_general(zb, zb, (((0,), (0,)), ((), ())),
                                  preferred_element_type=jnp.float32,
                                  precision=lax.Precision.DEFAULT)
    s_ref[0:1, :] += jnp.sum(zb, axis=0, keepdims=True)


def _p4(z1, w, b, din, dout):
    return pl.pallas_call(
        _p4_body,
        grid=(_NBLK,),
        in_specs=[pl.BlockSpec((_BN, din), lambda i: (i, 0)),
                  pl.BlockSpec((din, dout), lambda i: (0, 0)),
                  pl.BlockSpec((8, dout), lambda i: (0, 0))],
        out_specs=[pl.BlockSpec((_BN, dout), lambda i: (i, 0)),
                   pl.BlockSpec((dout, dout), lambda i: (0, 0)),
                   pl.BlockSpec((8, dout), lambda i: (0, 0))],
        out_shape=[jax.ShapeDtypeStruct((_NPAD, dout), jnp.float32),
                   jax.ShapeDtypeStruct((dout, dout), jnp.float32),
                   jax.ShapeDtypeStruct((8, dout), jnp.float32)],
    )(z1, w, jnp.broadcast_to(b[None, :], (8, dout)))


def _p5_body(z_ref, w3_ref, b3_ref, w4_ref, b4_ref, y0_ref, y1_ref, y2_ref, y3_ref):
    i = pl.program_id(0)
    z3 = jnp.maximum(_dot(z_ref[...], w3_ref[...], lax.Precision.DEFAULT)
                     + b3_ref[0:1, :], 0.0)
    y = _dot(z3, w4_ref[...], lax.Precision.DEFAULT) + b4_ref[0:1, :]
    row = jax.lax.broadcasted_iota(jnp.int32, (_BN, 1), 0) + i * _BN
    y = y * (row < _N).astype(jnp.float32)
    y0_ref[...] = y[:, 0:16]
    y1_ref[...] = y[:, 16:32]
    y2_ref[...] = y[:, 32:48]
    y3_ref[...] = y[:, 48:64]


def _p5(z2, w3, b3, w4, b4):
    return pl.pallas_call(
        _p5_body,
        grid=(_NBLK,),
        in_specs=[pl.BlockSpec((_BN, 128), lambda i: (i, 0)),
                  pl.BlockSpec((128, 256), lambda i: (0, 0)),
                  pl.BlockSpec((8, 256), lambda i: (0, 0)),
                  pl.BlockSpec((256, 64), lambda i: (0, 0)),
                  pl.BlockSpec((8, 64), lambda i: (0, 0))],
        out_specs=[pl.BlockSpec((_BN, 16), lambda i: (i, 0))] * 4,
        out_shape=[jax.ShapeDtypeStruct((_NPAD, 16), jnp.float32)] * 4,
    )(z2, w3, jnp.broadcast_to(b3[None, :], (8, 256)),
      w4, jnp.broadcast_to(b4[None, :], (8, 64)))


def _p6_body(t_ref, g_ref, o_ref):
    tT = lax.dot_general(jnp.eye(16, dtype=jnp.float32), t_ref[0, 0],
                         (((1,), (1,)), ((), ())), preferred_element_type=jnp.float32, precision=lax.Precision.HIGHEST)
    gT = lax.dot_general(jnp.eye(8, dtype=jnp.float32), g_ref[...],
                         (((1,), (1,)), ((), ())), preferred_element_type=jnp.float32, precision=lax.Precision.HIGHEST)
    cnt = jnp.maximum(gT[5:6, :], 1.0)
    o_ref[...] = (tT / cnt)[None]


def _p6(temp, grid6):
    nvb = _QH // _RB
    return pl.pallas_call(
        _p6_body,
        grid=(2, 2, 4, nvb),
        in_specs=[
            pl.BlockSpec((1, 1, _RB, 16),
                         lambda b, rh, cb, i: (b * 2 + rh, cb, i, 0)),
            pl.BlockSpec((_RB, 8),
                         lambda b, rh, cb, i: ((b * 2 + rh) * (_QH // _RB) + i, 0)),
        ],
        out_specs=pl.BlockSpec((1, 16, _RB),
                               lambda b, rh, cb, i: (b, cb, rh * nvb + i)),
        out_shape=jax.ShapeDtypeStruct((2, 64, 2 * _QH), jnp.float32),
    )(temp, grid6)


_NS = 16
_HALF = _NV // 2
_TPP = _NPAD // _NS
_NROW = _TPP // 128
_SDUM = 2048
_GDUM = 2048
_ZR = (_HALF + _SDUM) // _NS // 4


def _sca_body(keys_hbm, vals_hbm, zeros_hbm, grid6_hbm, gath_hbm,
              sbuf, kbuf, ibuf, vbuf, gbuf):
    c = lax.axis_index("c")
    s = lax.axis_index("s")
    base = s * _TPP
    cbase = c * _HALF
    lane = lax.broadcasted_iota(jnp.int32, (16,), 0)

    pltpu.sync_copy(zeros_hbm, sbuf.at[pl.ds(s * (4 * _ZR), 4 * _ZR)])

    pltpu.sync_copy(keys_hbm.at[pl.ds(base, _TPP)], kbuf)

    @pl.loop(0, _NROW)
    def _prep(g):
        for jj in range(8):
            v = g * 128 + jj * 16
            k = kbuf[pl.ds(v, 16)]
            local = k - cbase
            owned = (local >= 0) & (local < _HALF)
            sdum = _HALF + s * 128 + jj * 16 + lane
            ibuf[g, pl.ds(jj * 16, 16)] = jnp.where(owned, local, sdum)

    plsc.subcore_barrier()

    @pl.loop(0, _NROW)
    def _scat(g):
        pltpu.sync_copy(vals_hbm.at[pl.ds(base + g * 128, 128)], vbuf)
        pltpu.sync_copy(vbuf, sbuf.at[ibuf.at[g]], add=True)

    plsc.subcore_barrier()

    @pl.loop(0, _NROW)
    def _gath(g):
        pltpu.sync_copy(sbuf.at[ibuf.at[g]], gbuf)
        pltpu.sync_copy(gbuf, gath_hbm.at[c, pl.ds(base + g * 128, 128)])

    rows = _HALF // _NS
    pltpu.sync_copy(sbuf.at[pl.ds(s * rows, rows)],
                    grid6_hbm.at[pl.ds(cbase + s * rows, rows)])


@functools.lru_cache(maxsize=None)
def _sca():
    return pl.kernel(
        _sca_body,
        out_type=[jax.ShapeDtypeStruct((_NV, 8), jnp.float32),
                  jax.ShapeDtypeStruct((2, _NPAD, 8), jnp.float32)],
        mesh=plsc.VectorSubcoreMesh(core_axis_name="c", subcore_axis_name="s",
                                    num_cores=2, num_subcores=_NS),
        compiler_params=pltpu.CompilerParams(use_tc_tiling_on_sc=False),
        scratch_types=[
            pltpu.VMEM_SHARED((_HALF + _SDUM, 8), jnp.float32),
            pltpu.VMEM((_TPP,), jnp.int32),
            pltpu.VMEM((_NROW, 128), jnp.int32),
            pltpu.VMEM((128, 8), jnp.float32),
            pltpu.VMEM((128, 8), jnp.float32),
        ],
    )


_EROWS = _QH + 512
_EZR = _EROWS // _NS


def _sce_body(keys_hbm, y0_hbm, y1_hbm, y2_hbm, y3_hbm, zeros_hbm, temp_hbm,
              ebuf, kbuf, ibuf, vbuf, sem):
    c = lax.axis_index("c")
    s = lax.axis_index("s")
    base = s * _TPP
    lane = lax.broadcasted_iota(jnp.int32, (16,), 0)
    xrows = _QH // _NS
    y_hbms = (y0_hbm, y1_hbm, y2_hbm, y3_hbm)

    for h in range(2):
        rc = c * 2 + h
        rbase = rc * _QH

        for part in range(2):
            pltpu.sync_copy(keys_hbm.at[pl.ds(base + part * 6400, 6400)], kbuf)

            @pl.loop(0, _NROW // 2)
            def _prep(g):
                for jj in range(8):
                    v = g * 128 + jj * 16
                    k = kbuf[pl.ds(v, 16)]
                    local = k - rbase
                    owned = (local >= 0) & (local < _QH)
                    sdum = _QH + s * 32 + (jj % 2) * 16 + lane
                    ibuf[part * (_NROW // 2) + g, pl.ds(jj * 16, 16)] = \
                        jnp.where(owned, local, sdum)

        for cb in range(4):
            plsc.subcore_barrier()
            pltpu.sync_copy(zeros_hbm, ebuf.at[pl.ds(s * _EZR, _EZR)])
            plsc.subcore_barrier()
            y_hbm = y_hbms[cb]

            for q in range(10):
                pltpu.sync_copy(y_hbm.at[pl.ds(base + q * 1280, 1280)], vbuf)

                @pl.loop(0, 10)
                def _fire(g):
                    pltpu.async_copy(vbuf.at[pl.ds(g * 128, 128)],
                                     ebuf.at[ibuf.at[q * 10 + g]], sem, add=True)

                @pl.loop(0, 10)
                def _drain(g):
                    pltpu.make_async_copy(zeros_hbm.at[pl.ds(0, 128)],
                                          vbuf.at[pl.ds(0, 128)], sem).wait()

            plsc.subcore_barrier()
            pltpu.sync_copy(ebuf.at[pl.ds(s * xrows, xrows)],
                            temp_hbm.at[rc, cb, pl.ds(s * xrows, xrows)])


@functools.lru_cache(maxsize=None)
def _sce():
    return pl.kernel(
        _sce_body,
        out_type=jax.ShapeDtypeStruct((4, 4, _QH, 16), jnp.float32),
        mesh=plsc.VectorSubcoreMesh(core_axis_name="c", subcore_axis_name="s",
                                    num_cores=2, num_subcores=_NS),
        compiler_params=pltpu.CompilerParams(use_tc_tiling_on_sc=False),
        scratch_types=[
            pltpu.VMEM_SHARED((_EROWS, 16), jnp.float32),
            pltpu.VMEM((6400,), jnp.int32),
            pltpu.VMEM((_NROW, 128), jnp.int32),
            pltpu.VMEM((1280, 16), jnp.float32),
            pltpu.SemaphoreType.DMA,
        ],
    )


def _fold1(G, bn0_g, bn0_b, w1, b1, bn1_g, bn1_b):
    n = G[14, 14]
    sf = G[14, :14]
    f2 = G[:14, :14]
    mu = sf / n
    var0 = jnp.maximum(jnp.diag(f2) / n - mu * mu, 0.0)
    a0 = bn0_g / jnp.sqrt(var0 + _EPS)
    c0 = bn0_b - mu * a0
    A1 = a0[:, None] * w1
    d1 = c0 @ w1 + b1
    mu1 = mu @ A1 + d1
    Cf = f2 / n - mu[:, None] * mu[None, :]
    var1 = jnp.maximum(jnp.sum(A1 * (Cf @ A1), axis=0), 0.0)
    s1 = bn1_g / jnp.sqrt(var1 + _EPS)
    w1e = A1 * s1[None, :]
    b1e = (d1 - mu1) * s1 + bn1_b
    wa = jnp.zeros((16, 64), jnp.float32)
    wa = wa.at[:14, :].set(w1e).at[14, :].set(b1e)
    return wa


def _fold_mid(M, S, w, b, bng, bnb):
    n = jnp.float32(_N)
    muz = S[0] / n
    Cz = M / n - muz[:, None] * muz[None, :]
    muy = muz @ w + b
    var = jnp.maximum(jnp.sum(w * (Cz @ w), axis=0), 0.0)
    t = bng / jnp.sqrt(var + _EPS)
    return w * t[None, :], (b - muy) * t + bnb


def kernel(points, batch_size, bn0_g, bn0_b, lin1_w, lin1_b, bn1_g, bn1_b,
           lin2_w, lin2_b, bn2_g, bn2_b, lin3_w, lin3_b, bn3_g, bn3_b,
           lin4_w, lin4_b):
    f32 = jnp.float32
    pts_t = jnp.zeros((8, _NPAD), f32).at[:5, :_N].set(points.T)
    valsA = _p1(pts_t)
    rho0 = jnp.sqrt(points[:, 1] ** 2 + points[:, 2] ** 2)
    phi0 = jnp.arctan2(points[:, 2], points[:, 1])
    vr0 = jnp.clip(jnp.floor((rho0 - _PCR[0]) / _VOX[0]).astype(jnp.int32), 0, _GRID[0] - 1)
    vp0 = jnp.clip(jnp.floor((phi0 - _PCR[1]) / _VOX[1]).astype(jnp.int32), 0, _GRID[1] - 1)
    b0 = points[:, 0].astype(jnp.int32)
    key0 = (b0 * _GRID[0] + vr0) * _GRID[1] + vp0
    keys = jnp.full((_NPAD,), _SENT, jnp.int32).at[:_N].set(key0)

    zeros8 = jnp.zeros((4 * _ZR, 8), f32)
    grid6, gathered = _sca()(keys, valsA, zeros8)

    fch, G = _p2(pts_t, gathered, keys.reshape(_NBLK, 1, _BN))
    wa1 = _fold1(G, bn0_g, bn0_b, lin1_w, lin1_b, bn1_g, bn1_b)
    z1, M1, S1 = _p3(fch, wa1)
    w2e, b2e = _fold_mid(M1, S1, lin2_w, lin2_b, bn2_g, bn2_b)
    z2, M2, S2 = _p4(z1, w2e, b2e, 64, 128)
    w3e, b3e = _fold_mid(M2, S2, lin3_w, lin3_b, bn3_g, bn3_b)
    y0, y1, y2, y3 = _p5(z2, w3e, b3e, lin4_w, lin4_b)

    zeros16 = jnp.zeros((_EZR, 16), f32)
    temp = _sce()(keys, y0, y1, y2, y3, zeros16)

    out = _p6(temp, grid6)
    return out.reshape(2, 64, _GRID[0], _GRID[1])
